# baseline (speedup 1.0000x reference)
import functools

import jax
import jax.numpy as jnp
from jax import lax
from jax.experimental import pallas as pl
from jax.experimental.pallas import tpu as pltpu
from jax.experimental.pallas import tpu_sc as plsc

_NUM_CORES = 2
_NUM_SUBCORES = 16
_LANES = 16
_NW = _NUM_CORES * _NUM_SUBCORES


def _chunk_sizes(b_per_w):
    if b_per_w == 128:
        return [48, 32, 32, 16]
    n = max(1, b_per_w // 32)
    return [b_per_w // n] * n


def _make_gather(n_rows, n_sub, d, b):
    assert b % (8 * _NW) == 0
    b_per_w = b // _NW
    sizes = _chunk_sizes(b_per_w)
    n_chunks = len(sizes)
    offs = [sum(sizes[:i]) for i in range(n_chunks)]
    assert all(s % 8 == 0 for s in sizes) and sum(sizes) == b_per_w
    mesh = plsc.VectorSubcoreMesh(core_axis_name="c", subcore_axis_name="s")

    @functools.partial(
        pl.kernel,
        mesh=mesh,
        out_type=jax.ShapeDtypeStruct((b, d), jnp.float32),
        scratch_types=[
            pltpu.VMEM((b_per_w,), jnp.int32),
            pltpu.VMEM((b_per_w,), jnp.int32),
            pltpu.VMEM((b_per_w,), jnp.int32),
            pltpu.VMEM((b_per_w, d), jnp.float32),
            [pltpu.SemaphoreType.DMA] * n_chunks,
            [pltpu.SemaphoreType.DMA] * n_chunks,
            [pltpu.SemaphoreType.DMA] * n_chunks,
            [pltpu.SemaphoreType.DMA] * n_chunks,
        ],
    )
    def gather_kernel(table_hbm, ds_hbm, sub_hbm, out_hbm,
                      ds_v, sub_v, flat_v, rows_v,
                      sem_ds, sem_sub, gsems, ssems):
        wid = lax.axis_index("s") * _NUM_CORES + lax.axis_index("c")
        base = wid * b_per_w
        idx_cps = []
        for c in range(n_chunks):
            sl_h = pl.ds(base + offs[c], sizes[c])
            sl_v = pl.ds(offs[c], sizes[c])
            idx_cps.append((
                pltpu.async_copy(ds_hbm.at[sl_h], ds_v.at[sl_v], sem_ds[c]),
                pltpu.async_copy(sub_hbm.at[sl_h], sub_v.at[sl_v], sem_sub[c]),
            ))
        gathers = []
        for c in range(n_chunks):
            idx_cps[c][0].wait()
            idx_cps[c][1].wait()
            for i in range(sizes[c] // _LANES):
                sl = pl.ds(offs[c] + i * _LANES, _LANES)
                flat_v[sl] = ds_v[sl] * n_sub + sub_v[sl]
            gathers.append(pltpu.async_copy(
                table_hbm.at[flat_v.at[pl.ds(offs[c], sizes[c])]],
                rows_v.at[pl.ds(offs[c], sizes[c])], gsems[c]))
        stores = []
        for c in range(n_chunks):
            gathers[c].wait()
            stores.append(pltpu.async_copy(
                rows_v.at[pl.ds(offs[c], sizes[c])],
                out_hbm.at[pl.ds(base + offs[c], sizes[c])], ssems[c]))
        for c in range(n_chunks):
            stores[c].wait()

    return gather_kernel


def kernel(table, dataset_idx, subject_idx):
    n_ds, n_sub, d = table.shape
    (b,) = dataset_idx.shape
    flat_table = table.reshape(n_ds * n_sub, d)
    fn = _make_gather(n_ds * n_sub, n_sub, d, b)
    return fn(flat_table,
              dataset_idx.astype(jnp.int32),
              subject_idx.astype(jnp.int32))


# speedup vs baseline: 1.0035x; 1.0035x over previous
import functools

import jax
import jax.numpy as jnp
from jax import lax
from jax.experimental import pallas as pl
from jax.experimental.pallas import tpu as pltpu
from jax.experimental.pallas import tpu_sc as plsc

_NUM_CORES = 2
_NUM_SUBCORES = 16
_LANES = 16
_NW = _NUM_CORES * _NUM_SUBCORES


def _flat_idx_tc(ds_ref, sub_ref, flat_ref, n_sub):
    flat_ref[...] = ds_ref[...] * n_sub + sub_ref[...]


def _chunk_sizes(b_per_w):
    if b_per_w == 128:
        return [48, 32, 32, 16]
    n = max(1, b_per_w // 32)
    return [b_per_w // n] * n


def _make_gather(n_rows, d, b):
    assert b % (8 * _NW) == 0
    b_per_w = b // _NW
    sizes = _chunk_sizes(b_per_w)
    n_chunks = len(sizes)
    offs = [sum(sizes[:i]) for i in range(n_chunks)]
    assert all(s % 8 == 0 for s in sizes) and sum(sizes) == b_per_w
    mesh = plsc.VectorSubcoreMesh(core_axis_name="c", subcore_axis_name="s")

    @functools.partial(
        pl.kernel,
        mesh=mesh,
        out_type=jax.ShapeDtypeStruct((b, d), jnp.float32),
        scratch_types=[
            pltpu.VMEM((b_per_w,), jnp.int32),
            pltpu.VMEM((b_per_w, d), jnp.float32),
            [pltpu.SemaphoreType.DMA] * n_chunks,
            [pltpu.SemaphoreType.DMA] * n_chunks,
            [pltpu.SemaphoreType.DMA] * n_chunks,
        ],
    )
    def gather_kernel(table_hbm, flat_hbm, out_hbm,
                      flat_v, rows_v, isems, gsems, ssems):
        wid = lax.axis_index("s") * _NUM_CORES + lax.axis_index("c")
        base = wid * b_per_w
        idx_cps = []
        for c in range(n_chunks):
            idx_cps.append(pltpu.async_copy(
                flat_hbm.at[pl.ds(base + offs[c], sizes[c])],
                flat_v.at[pl.ds(offs[c], sizes[c])], isems[c]))
        gathers = []
        for c in range(n_chunks):
            idx_cps[c].wait()
            gathers.append(pltpu.async_copy(
                table_hbm.at[flat_v.at[pl.ds(offs[c], sizes[c])]],
                rows_v.at[pl.ds(offs[c], sizes[c])], gsems[c]))
        stores = []
        for c in range(n_chunks):
            gathers[c].wait()
            stores.append(pltpu.async_copy(
                rows_v.at[pl.ds(offs[c], sizes[c])],
                out_hbm.at[pl.ds(base + offs[c], sizes[c])], ssems[c]))
        for c in range(n_chunks):
            stores[c].wait()

    return gather_kernel


def kernel(table, dataset_idx, subject_idx):
    n_ds, n_sub, d = table.shape
    (b,) = dataset_idx.shape
    flat_table = table.reshape(n_ds * n_sub, d)
    ds2 = dataset_idx.astype(jnp.int32).reshape(b // 128, 128)
    sub2 = subject_idx.astype(jnp.int32).reshape(b // 128, 128)
    flat2 = pl.pallas_call(
        functools.partial(_flat_idx_tc, n_sub=n_sub),
        out_shape=jax.ShapeDtypeStruct((b // 128, 128), jnp.int32),
    )(ds2, sub2)
    flat = flat2.reshape(b)
    fn = _make_gather(n_ds * n_sub, d, b)
    return fn(flat_table, flat)
